# baseline (speedup 1.0000x reference)
import functools

import jax
import jax.numpy as jnp
from jax import lax
from jax.experimental import pallas as pl
from jax.experimental.pallas import tpu as pltpu
from jax.experimental.pallas import tpu_sc as plsc

VOCAB = 1000000
DIM = 64
BATCH = 16384
HIST = 200

NC = 2
NS = 16
NW = NC * NS

B = BATCH * HIST
BPW = B // NW
CHUNK = 128
NCHUNK = BPW // CHUNK

_mesh = plsc.VectorSubcoreMesh(core_axis_name="c", subcore_axis_name="s")


@functools.partial(
    pl.kernel,
    out_type=jax.ShapeDtypeStruct((NW, NCHUNK, CHUNK, DIM), jnp.float32),
    mesh=_mesh,
    scratch_types=[
        pltpu.VMEM((NCHUNK, CHUNK), jnp.int32),
        pltpu.VMEM((CHUNK, DIM), jnp.float32),
        pltpu.SemaphoreType.DMA,
    ],
    compiler_params=pltpu.CompilerParams(use_tc_tiling_on_sc=False),
)
def _embed(x_hbm, w_hbm, out_hbm, idx_v, rows_v, sem):
    wid = lax.axis_index("s") * NC + lax.axis_index("c")
    pltpu.sync_copy(x_hbm.at[wid], idx_v)

    def step(j, carry):
        pltpu.async_copy(w_hbm.at[idx_v.at[j]], rows_v, sem).wait()
        pltpu.sync_copy(rows_v, out_hbm.at[wid, j])
        return carry

    lax.fori_loop(0, NCHUNK, step, 0)


def kernel(x, weight):
    xf = x.reshape(NW, NCHUNK, CHUNK).astype(jnp.int32)
    out = _embed(xf, weight)
    return out.reshape(BATCH, HIST, DIM)


# speedup vs baseline: 1.1528x; 1.1528x over previous
import functools

import jax
import jax.numpy as jnp
from jax import lax
from jax.experimental import pallas as pl
from jax.experimental.pallas import tpu as pltpu
from jax.experimental.pallas import tpu_sc as plsc

VOCAB = 1000000
DIM = 64
BATCH = 16384
HIST = 200

NC = 2
NS = 16
NW = NC * NS

B = BATCH * HIST
BPW = B // NW
CHUNK = 128
NCHUNK = BPW // CHUNK
NBUF = 2

_mesh = plsc.VectorSubcoreMesh(core_axis_name="c", subcore_axis_name="s")


@functools.partial(
    pl.kernel,
    out_type=jax.ShapeDtypeStruct((NW, NCHUNK, CHUNK, DIM), jnp.float32),
    mesh=_mesh,
    scratch_types=[
        pltpu.VMEM((NCHUNK, CHUNK), jnp.int32),
        pltpu.VMEM((NBUF, CHUNK, DIM), jnp.float32),
        pltpu.SemaphoreType.DMA((NBUF,)),
        pltpu.SemaphoreType.DMA((NBUF,)),
    ],
    compiler_params=pltpu.CompilerParams(use_tc_tiling_on_sc=False),
)
def _embed(x_hbm, w_hbm, out_hbm, idx_v, rows_v, gsem, ssem):
    wid = lax.axis_index("s") * NC + lax.axis_index("c")
    pltpu.sync_copy(x_hbm.at[wid], idx_v)

    for b in range(NBUF):
        pltpu.async_copy(w_hbm.at[idx_v.at[b]], rows_v.at[b], gsem.at[b])

    def outer(i, carry):
        for b in range(NBUF):
            j = i * NBUF + b
            pltpu.make_async_copy(w_hbm.at[idx_v.at[j]], rows_v.at[b],
                                  gsem.at[b]).wait()
            pltpu.async_copy(rows_v.at[b], out_hbm.at[wid, j], ssem.at[b])
            pltpu.make_async_copy(rows_v.at[b], out_hbm.at[wid, j],
                                  ssem.at[b]).wait()
            pltpu.async_copy(w_hbm.at[idx_v.at[j + NBUF]], rows_v.at[b],
                             gsem.at[b])
        return carry

    lax.fori_loop(0, NCHUNK // NBUF - 1, outer, 0)

    for b in range(NBUF):
        j = NCHUNK - NBUF + b
        pltpu.make_async_copy(w_hbm.at[idx_v.at[j]], rows_v.at[b],
                              gsem.at[b]).wait()
        pltpu.async_copy(rows_v.at[b], out_hbm.at[wid, j], ssem.at[b])
    for b in range(NBUF):
        j = NCHUNK - NBUF + b
        pltpu.make_async_copy(rows_v.at[b], out_hbm.at[wid, j],
                              ssem.at[b]).wait()


def kernel(x, weight):
    xf = x.reshape(NW, NCHUNK, CHUNK).astype(jnp.int32)
    out = _embed(xf, weight)
    return out.reshape(BATCH, HIST, DIM)


# speedup vs baseline: 1.1895x; 1.0318x over previous
import functools

import jax
import jax.numpy as jnp
from jax import lax
from jax.experimental import pallas as pl
from jax.experimental.pallas import tpu as pltpu
from jax.experimental.pallas import tpu_sc as plsc

VOCAB = 1000000
DIM = 64
BATCH = 16384
HIST = 200

NC = 2
NS = 16
NW = NC * NS

B = BATCH * HIST
BPW = B // NW
CHUNK = 512
NCHUNK = BPW // CHUNK
NBUF = 2

_mesh = plsc.VectorSubcoreMesh(core_axis_name="c", subcore_axis_name="s")


@functools.partial(
    pl.kernel,
    out_type=jax.ShapeDtypeStruct((NW, NCHUNK, CHUNK, DIM), jnp.float32),
    mesh=_mesh,
    scratch_types=[
        pltpu.VMEM((NBUF, CHUNK), jnp.int32),
        pltpu.VMEM((NBUF, CHUNK, DIM), jnp.float32),
        pltpu.SemaphoreType.DMA((NBUF,)),
        pltpu.SemaphoreType.DMA((NBUF,)),
        pltpu.SemaphoreType.DMA((NBUF,)),
    ],
    compiler_params=pltpu.CompilerParams(use_tc_tiling_on_sc=False),
)
def _embed(x_hbm, w_hbm, out_hbm, idx_v, rows_v, isem, gsem, ssem):
    wid = lax.axis_index("s") * NC + lax.axis_index("c")

    for b in range(NBUF):
        pltpu.async_copy(x_hbm.at[wid, b], idx_v.at[b], isem.at[b])
    for b in range(NBUF):
        pltpu.make_async_copy(x_hbm.at[wid, b], idx_v.at[b], isem.at[b]).wait()
        pltpu.async_copy(w_hbm.at[idx_v.at[b]], rows_v.at[b], gsem.at[b])

    def outer(i, carry):
        for b in range(NBUF):
            j = i * NBUF + b
            pltpu.make_async_copy(w_hbm.at[idx_v.at[b]], rows_v.at[b],
                                  gsem.at[b]).wait()
            pltpu.async_copy(rows_v.at[b], out_hbm.at[wid, j], ssem.at[b])
            pltpu.async_copy(x_hbm.at[wid, j + NBUF], idx_v.at[b], isem.at[b])
            pltpu.make_async_copy(rows_v.at[b], out_hbm.at[wid, j],
                                  ssem.at[b]).wait()
            pltpu.make_async_copy(x_hbm.at[wid, j + NBUF], idx_v.at[b],
                                  isem.at[b]).wait()
            pltpu.async_copy(w_hbm.at[idx_v.at[b]], rows_v.at[b], gsem.at[b])
        return carry

    lax.fori_loop(0, NCHUNK // NBUF - 1, outer, 0)

    for b in range(NBUF):
        j = NCHUNK - NBUF + b
        pltpu.make_async_copy(w_hbm.at[idx_v.at[b]], rows_v.at[b],
                              gsem.at[b]).wait()
        pltpu.async_copy(rows_v.at[b], out_hbm.at[wid, j], ssem.at[b])
    for b in range(NBUF):
        j = NCHUNK - NBUF + b
        pltpu.make_async_copy(rows_v.at[b], out_hbm.at[wid, j],
                              ssem.at[b]).wait()


def kernel(x, weight):
    xf = x.reshape(NW, NCHUNK, CHUNK).astype(jnp.int32)
    out = _embed(xf, weight)
    return out.reshape(BATCH, HIST, DIM)
